# baseline (speedup 1.0000x reference)
import functools

import jax
import jax.numpy as jnp
from jax import lax
from jax.experimental import pallas as pl
from jax.experimental.pallas import tpu as pltpu
from jax.experimental.pallas import tpu_sc as plsc

BATCH = 16384
EMB = 64
NF_PAD = 8

_NC, _NS = 2, 16
_NW = _NC * _NS
_BPW = BATCH // _NW
_CH = 128
_NCH = _BPW // _CH


def _sc_gather_body(utab, itab, uidx, iidx, uout, iout,
                    uidx_v, iidx_v, rows_v, sem):
    wid = lax.axis_index("s") * _NC + lax.axis_index("c")
    base = wid * _BPW
    jbase = wid * _NCH
    pltpu.sync_copy(uidx.at[pl.ds(jbase, _NCH)], uidx_v)
    pltpu.sync_copy(iidx.at[pl.ds(jbase, _NCH)], iidx_v)
    copies = [
        pltpu.async_copy(utab.at[uidx_v.at[j]],
                         rows_v.at[pl.ds(j * _CH, _CH)], sem)
        for j in range(_NCH)
    ]
    for c in copies:
        c.wait()
    pltpu.sync_copy(rows_v, uout.at[pl.ds(base, _BPW)])
    copies = [
        pltpu.async_copy(itab.at[iidx_v.at[j]],
                         rows_v.at[pl.ds(j * _CH, _CH)], sem)
        for j in range(_NCH)
    ]
    for c in copies:
        c.wait()
    pltpu.sync_copy(rows_v, iout.at[pl.ds(base, _BPW)])


@functools.cache
def _sc_gather():
    return pl.kernel(
        _sc_gather_body,
        out_type=(
            jax.ShapeDtypeStruct((BATCH, 2 * EMB), jnp.float32),
            jax.ShapeDtypeStruct((BATCH, 2 * EMB), jnp.float32),
        ),
        mesh=plsc.VectorSubcoreMesh(core_axis_name="c", subcore_axis_name="s",
                                    num_cores=_NC, num_subcores=_NS),
        scratch_types=[
            pltpu.VMEM((_NCH, _CH), jnp.int32),
            pltpu.VMEM((_NCH, _CH), jnp.int32),
            pltpu.VMEM((_BPW, 2 * EMB), jnp.float32),
            pltpu.SemaphoreType.DMA,
        ],
    )


def _mlp_body(u2_ref, i2_ref, uf_ref, if_ref, nf_ref, wnum_ref, bnum_ref,
              wout_ref, bout_ref, o_ref):
    uf = uf_ref[:]
    itf = if_ref[:]
    upar = uf - 2.0 * jnp.floor(uf * 0.5)
    ipar = itf - 2.0 * jnp.floor(itf * 0.5)
    u2 = u2_ref[:]
    i2 = i2_ref[:]
    u = jnp.where(upar > 0.5, u2[:, EMB:2 * EMB], u2[:, 0:EMB])
    i = jnp.where(ipar > 0.5, i2[:, EMB:2 * EMB], i2[:, 0:EMB])
    y1 = jnp.dot(nf_ref[:], wnum_ref[:], preferred_element_type=jnp.float32)
    y1 = jnp.maximum(y1 + bnum_ref[:], 0.0)
    wout = wout_ref[:]
    acc = jnp.dot(u, wout[0:EMB], preferred_element_type=jnp.float32)
    acc += jnp.dot(i, wout[EMB:2 * EMB], preferred_element_type=jnp.float32)
    acc += jnp.dot(y1, wout[2 * EMB:3 * EMB],
                   preferred_element_type=jnp.float32)
    o_ref[:] = jnp.maximum(acc + bout_ref[:], 0.0)


def _mlp(u2, i2, uf, itf, nf_pad, wnum_pad, bnum, wout, bout, block_b=2048):
    grid = (BATCH // block_b,)
    return pl.pallas_call(
        _mlp_body,
        grid=grid,
        in_specs=[
            pl.BlockSpec((block_b, 2 * EMB), lambda g: (g, 0)),
            pl.BlockSpec((block_b, 2 * EMB), lambda g: (g, 0)),
            pl.BlockSpec((block_b, 1), lambda g: (g, 0)),
            pl.BlockSpec((block_b, 1), lambda g: (g, 0)),
            pl.BlockSpec((block_b, NF_PAD), lambda g: (g, 0)),
            pl.BlockSpec((NF_PAD, EMB), lambda g: (0, 0)),
            pl.BlockSpec((1, EMB), lambda g: (0, 0)),
            pl.BlockSpec((3 * EMB, EMB), lambda g: (0, 0)),
            pl.BlockSpec((1, EMB), lambda g: (0, 0)),
        ],
        out_specs=pl.BlockSpec((block_b, EMB), lambda g: (g, 0)),
        out_shape=jax.ShapeDtypeStruct((BATCH, EMB), jnp.float32),
    )(u2, i2, uf, itf, nf_pad, wnum_pad, bnum, wout, bout)


def kernel(inputs, user_table, item_table, W_num, b_num, W_out, b_out):
    user_ids = inputs[:, 0].astype(jnp.int32)
    item_ids = inputs[:, 1].astype(jnp.int32)
    upack = (user_ids >> 1).reshape(_NW * _NCH, _CH)
    ipack = (item_ids >> 1).reshape(_NW * _NCH, _CH)
    utab2 = user_table.reshape(-1, 2 * EMB)
    itab2 = item_table.reshape(-1, 2 * EMB)
    nf_pad = jnp.pad(inputs[:, 2:], ((0, 0), (0, NF_PAD - 5)))
    wnum_pad = jnp.pad(W_num, ((0, NF_PAD - 5), (0, 0)))
    u2, i2 = _sc_gather()(utab2, itab2, upack, ipack)
    return _mlp(u2, i2, inputs[:, 0:1], inputs[:, 1:2], nf_pad, wnum_pad,
                b_num.reshape(1, EMB), W_out, b_out.reshape(1, EMB))


# speedup vs baseline: 1.5862x; 1.5862x over previous
import functools

import jax
import jax.numpy as jnp
from jax import lax
from jax.experimental import pallas as pl
from jax.experimental.pallas import tpu as pltpu
from jax.experimental.pallas import tpu_sc as plsc

BATCH = 16384
EMB = 64
NF_PAD = 8

_NC, _NS = 2, 16
_NW = _NC * _NS
_BPW = BATCH // _NW
_CH = 64
_NCH = _BPW // _CH
_L = 16


def _fire_rows(tab, idx_v, out_v, sem):
    def body(g, carry):
        vec = idx_v[pl.ds(g * _L, _L)]
        for k in range(_L):
            t = vec[k]
            pltpu.async_copy(tab.at[pl.ds(t, 1)],
                             out_v.at[pl.ds(g * _L + k, 1)], sem)
        return carry

    lax.fori_loop(0, _BPW // _L, body, 0)


def _sc_gather_body(utab, itab, uidx, iidx, uout, iout,
                    uidx_v, iidx_v, out_v, sem):
    wid = lax.axis_index("s") * _NC + lax.axis_index("c")
    base = wid * _BPW
    pltpu.sync_copy(uidx.at[pl.ds(base, _BPW)], uidx_v)
    pltpu.sync_copy(iidx.at[pl.ds(base, _BPW)], iidx_v)
    _fire_rows(utab, uidx_v, out_v, sem)
    pltpu.make_async_copy(utab.at[pl.ds(0, _BPW)], out_v, sem).wait()
    pltpu.sync_copy(out_v, uout.at[pl.ds(base, _BPW)])
    _fire_rows(itab, iidx_v, out_v, sem)
    pltpu.make_async_copy(itab.at[pl.ds(0, _BPW)], out_v, sem).wait()
    pltpu.sync_copy(out_v, iout.at[pl.ds(base, _BPW)])


@functools.cache
def _sc_gather():
    return pl.kernel(
        _sc_gather_body,
        out_type=(
            jax.ShapeDtypeStruct((BATCH, EMB), jnp.float32),
            jax.ShapeDtypeStruct((BATCH, EMB), jnp.float32),
        ),
        mesh=plsc.VectorSubcoreMesh(core_axis_name="c", subcore_axis_name="s",
                                    num_cores=_NC, num_subcores=_NS),
        scratch_types=[
            pltpu.VMEM((_BPW,), jnp.int32),
            pltpu.VMEM((_BPW,), jnp.int32),
            pltpu.VMEM((_BPW, EMB), jnp.float32),
            pltpu.SemaphoreType.DMA,
        ],
    )


def _mlp_body(u_ref, i_ref, nf_ref, wnum_ref, bnum_ref, wout_ref, bout_ref,
              o_ref):
    y1 = jnp.dot(nf_ref[:], wnum_ref[:], preferred_element_type=jnp.float32)
    y1 = jnp.maximum(y1 + bnum_ref[:], 0.0)
    wout = wout_ref[:]
    acc = jnp.dot(u_ref[:], wout[0:EMB], preferred_element_type=jnp.float32)
    acc += jnp.dot(i_ref[:], wout[EMB:2 * EMB],
                   preferred_element_type=jnp.float32)
    acc += jnp.dot(y1, wout[2 * EMB:3 * EMB],
                   preferred_element_type=jnp.float32)
    o_ref[:] = jnp.maximum(acc + bout_ref[:], 0.0)


def _mlp(u, i, nf_pad, wnum_pad, bnum, wout, bout, block_b=2048):
    grid = (BATCH // block_b,)
    return pl.pallas_call(
        _mlp_body,
        grid=grid,
        in_specs=[
            pl.BlockSpec((block_b, EMB), lambda g: (g, 0)),
            pl.BlockSpec((block_b, EMB), lambda g: (g, 0)),
            pl.BlockSpec((block_b, NF_PAD), lambda g: (g, 0)),
            pl.BlockSpec((NF_PAD, EMB), lambda g: (0, 0)),
            pl.BlockSpec((1, EMB), lambda g: (0, 0)),
            pl.BlockSpec((3 * EMB, EMB), lambda g: (0, 0)),
            pl.BlockSpec((1, EMB), lambda g: (0, 0)),
        ],
        out_specs=pl.BlockSpec((block_b, EMB), lambda g: (g, 0)),
        out_shape=jax.ShapeDtypeStruct((BATCH, EMB), jnp.float32),
    )(u, i, nf_pad, wnum_pad, bnum, wout, bout)


def kernel(inputs, user_table, item_table, W_num, b_num, W_out, b_out):
    user_ids = inputs[:, 0].astype(jnp.int32)
    item_ids = inputs[:, 1].astype(jnp.int32)
    nf_pad = jnp.pad(inputs[:, 2:], ((0, 0), (0, NF_PAD - 5)))
    wnum_pad = jnp.pad(W_num, ((0, NF_PAD - 5), (0, 0)))
    u_rows, i_rows = _sc_gather()(user_table, item_table, user_ids, item_ids)
    return _mlp(u_rows, i_rows, nf_pad, wnum_pad,
                b_num.reshape(1, EMB), W_out, b_out.reshape(1, EMB))


# speedup vs baseline: 2.7520x; 1.7350x over previous
import functools

import jax
import jax.numpy as jnp
from jax import lax
from jax.experimental import pallas as pl
from jax.experimental.pallas import tpu as pltpu
from jax.experimental.pallas import tpu_sc as plsc

BATCH = 16384
EMB = 64
NF_PAD = 8

_NC, _NS = 2, 16
_NW = _NC * _NS
_BPW = BATCH // _NW
_L = 16
_RING = 4
_ROWS = _BPW // _L


def _gather_blocks(tabT, idx2_v, out_v, ring, sem):
    iota = lax.iota(jnp.int32, _L)

    def fire(id_scalar, slot):
        b0 = pl.multiple_of((id_scalar >> 7) * 128, 128)
        pltpu.async_copy(tabT.at[:, pl.ds(b0, 128)], ring.at[slot], sem)

    def wait_slot(slot):
        pltpu.make_async_copy(tabT.at[:, pl.ds(0, 128)], ring.at[slot],
                              sem).wait()

    def extract(id_scalar, pos, slot):
        lane = jnp.full((_L,), id_scalar & 127, jnp.int32)
        buf = ring.at[slot]
        for k in range(EMB // _L):
            vals = plsc.load_gather(buf, [iota + _L * k, lane])
            out_v[pos, pl.ds(_L * k, _L)] = vals

    def body(w, carry):
        vec = idx2_v[w]
        vecp = idx2_v[jnp.maximum(w - 1, 0)]
        for j in range(_L):
            slot = j % _RING
            if j < _RING:
                @pl.when(w > 0)
                def _():
                    wait_slot(slot)
                    extract(vecp[j + _L - _RING], w * _L + j - _RING, slot)
            else:
                wait_slot(slot)
                extract(vec[j - _RING], w * _L + j - _RING, slot)
            fire(vec[j], slot)
        return carry

    lax.fori_loop(0, _ROWS, body, 0)
    vec_last = idx2_v[_ROWS - 1]
    for j in range(_L - _RING, _L):
        slot = j % _RING
        wait_slot(slot)
        extract(vec_last[j], (_ROWS - 1) * _L + j, slot)


def _sc_gather_body(utabT, itabT, uidx, iidx, uout, iout,
                    uidx2_v, iidx2_v, out_v, ring, sem):
    wid = lax.axis_index("s") * _NC + lax.axis_index("c")
    base = wid * _BPW
    pltpu.sync_copy(uidx.at[pl.ds(wid * _ROWS, _ROWS)], uidx2_v)
    pltpu.sync_copy(iidx.at[pl.ds(wid * _ROWS, _ROWS)], iidx2_v)
    _gather_blocks(utabT, uidx2_v, out_v, ring, sem)
    pltpu.sync_copy(out_v, uout.at[pl.ds(base, _BPW)])
    _gather_blocks(itabT, iidx2_v, out_v, ring, sem)
    pltpu.sync_copy(out_v, iout.at[pl.ds(base, _BPW)])


@functools.cache
def _sc_gather():
    return pl.kernel(
        _sc_gather_body,
        out_type=(
            jax.ShapeDtypeStruct((BATCH, EMB), jnp.float32),
            jax.ShapeDtypeStruct((BATCH, EMB), jnp.float32),
        ),
        mesh=plsc.VectorSubcoreMesh(core_axis_name="c", subcore_axis_name="s",
                                    num_cores=_NC, num_subcores=_NS),
        compiler_params=pltpu.CompilerParams(needs_layout_passes=False),
        scratch_types=[
            pltpu.VMEM((_ROWS, _L), jnp.int32),
            pltpu.VMEM((_ROWS, _L), jnp.int32),
            pltpu.VMEM((_BPW, EMB), jnp.float32),
            pltpu.VMEM((_RING, EMB, 128), jnp.float32),
            pltpu.SemaphoreType.DMA,
        ],
    )


def _mlp_body(u_ref, i_ref, nf_ref, wnum_ref, bnum_ref, wout_ref, bout_ref,
              o_ref):
    y1 = jnp.dot(nf_ref[:], wnum_ref[:], preferred_element_type=jnp.float32)
    y1 = jnp.maximum(y1 + bnum_ref[:], 0.0)
    wout = wout_ref[:]
    acc = jnp.dot(u_ref[:], wout[0:EMB], preferred_element_type=jnp.float32)
    acc += jnp.dot(i_ref[:], wout[EMB:2 * EMB],
                   preferred_element_type=jnp.float32)
    acc += jnp.dot(y1, wout[2 * EMB:3 * EMB],
                   preferred_element_type=jnp.float32)
    o_ref[:] = jnp.maximum(acc + bout_ref[:], 0.0)


def _mlp(u, i, nf_pad, wnum_pad, bnum, wout, bout, block_b=2048):
    grid = (BATCH // block_b,)
    return pl.pallas_call(
        _mlp_body,
        grid=grid,
        in_specs=[
            pl.BlockSpec((block_b, EMB), lambda g: (g, 0)),
            pl.BlockSpec((block_b, EMB), lambda g: (g, 0)),
            pl.BlockSpec((block_b, NF_PAD), lambda g: (g, 0)),
            pl.BlockSpec((NF_PAD, EMB), lambda g: (0, 0)),
            pl.BlockSpec((1, EMB), lambda g: (0, 0)),
            pl.BlockSpec((3 * EMB, EMB), lambda g: (0, 0)),
            pl.BlockSpec((1, EMB), lambda g: (0, 0)),
        ],
        out_specs=pl.BlockSpec((block_b, EMB), lambda g: (g, 0)),
        out_shape=jax.ShapeDtypeStruct((BATCH, EMB), jnp.float32),
    )(u, i, nf_pad, wnum_pad, bnum, wout, bout)


def kernel(inputs, user_table, item_table, W_num, b_num, W_out, b_out):
    user_ids = inputs[:, 0].astype(jnp.int32)
    item_ids = inputs[:, 1].astype(jnp.int32)
    uidx2 = user_ids.reshape(_NW * _ROWS, _L)
    iidx2 = item_ids.reshape(_NW * _ROWS, _L)
    nf_pad = jnp.pad(inputs[:, 2:], ((0, 0), (0, NF_PAD - 5)))
    wnum_pad = jnp.pad(W_num, ((0, NF_PAD - 5), (0, 0)))
    u_rows, i_rows = _sc_gather()(user_table.T, item_table.T, uidx2, iidx2)
    return _mlp(u_rows, i_rows, nf_pad, wnum_pad,
                b_num.reshape(1, EMB), W_out, b_out.reshape(1, EMB))


# speedup vs baseline: 2.7525x; 1.0002x over previous
import functools

import jax
import jax.numpy as jnp
from jax import lax
from jax.experimental import pallas as pl
from jax.experimental.pallas import tpu as pltpu
from jax.experimental.pallas import tpu_sc as plsc

BATCH = 16384
EMB = 64
NF_PAD = 8

_NC, _NS = 2, 16
_NW = _NC * _NS
_BPW = BATCH // _NW
_L = 16
_RING = 4
_ROWS = _BPW // _L


def _gather_blocks(tabT, idx2_v, out_v, ring, sem):
    iota = lax.iota(jnp.int32, _L)
    tabT3 = tabT.reshape(EMB // 8, 8, tabT.shape[1])

    def fire(id_scalar, slot):
        b0 = pl.multiple_of((id_scalar >> 7) * 128, 128)
        pltpu.async_copy(tabT3.at[:, :, pl.ds(b0, 128)], ring.at[slot], sem)

    def wait_slot(slot):
        pltpu.make_async_copy(tabT3.at[:, :, pl.ds(0, 128)], ring.at[slot],
                              sem).wait()

    def extract(id_scalar, pos, slot):
        lane = jnp.full((_L,), id_scalar & 127, jnp.int32)
        buf = ring.at[slot].reshape(EMB, 128)
        for k in range(EMB // _L):
            vals = plsc.load_gather(buf, [iota + _L * k, lane])
            out_v[pos, pl.ds(_L * k, _L)] = vals

    def body(w, carry):
        vec = idx2_v[w]
        vecp = idx2_v[jnp.maximum(w - 1, 0)]
        for j in range(_L):
            slot = j % _RING
            if j < _RING:
                @pl.when(w > 0)
                def _():
                    wait_slot(slot)
                    extract(vecp[j + _L - _RING], w * _L + j - _RING, slot)
            else:
                wait_slot(slot)
                extract(vec[j - _RING], w * _L + j - _RING, slot)
            fire(vec[j], slot)
        return carry

    lax.fori_loop(0, _ROWS, body, 0)
    vec_last = idx2_v[_ROWS - 1]
    for j in range(_L - _RING, _L):
        slot = j % _RING
        wait_slot(slot)
        extract(vec_last[j], (_ROWS - 1) * _L + j, slot)


def _sc_gather_body(utabT, itabT, uidx, iidx, uout, iout,
                    uidx2_v, iidx2_v, out_v, ring, sem):
    wid = lax.axis_index("s") * _NC + lax.axis_index("c")
    base = wid * _BPW
    pltpu.sync_copy(uidx.at[pl.ds(wid * _ROWS, _ROWS)], uidx2_v)
    pltpu.sync_copy(iidx.at[pl.ds(wid * _ROWS, _ROWS)], iidx2_v)
    _gather_blocks(utabT, uidx2_v, out_v, ring, sem)
    pltpu.sync_copy(out_v, uout.at[pl.ds(base, _BPW)])
    _gather_blocks(itabT, iidx2_v, out_v, ring, sem)
    pltpu.sync_copy(out_v, iout.at[pl.ds(base, _BPW)])


@functools.cache
def _sc_gather():
    return pl.kernel(
        _sc_gather_body,
        out_type=(
            jax.ShapeDtypeStruct((BATCH, EMB), jnp.float32),
            jax.ShapeDtypeStruct((BATCH, EMB), jnp.float32),
        ),
        mesh=plsc.VectorSubcoreMesh(core_axis_name="c", subcore_axis_name="s",
                                    num_cores=_NC, num_subcores=_NS),
        compiler_params=pltpu.CompilerParams(needs_layout_passes=False),
        scratch_types=[
            pltpu.VMEM((_ROWS, _L), jnp.int32),
            pltpu.VMEM((_ROWS, _L), jnp.int32),
            pltpu.VMEM((_BPW, EMB), jnp.float32),
            pltpu.VMEM((_RING, EMB // 8, 8, 128), jnp.float32),
            pltpu.SemaphoreType.DMA,
        ],
    )


def _mlp_body(u_ref, i_ref, nf_ref, wnum_ref, bnum_ref, wout_ref, bout_ref,
              o_ref):
    y1 = jnp.dot(nf_ref[:], wnum_ref[:], preferred_element_type=jnp.float32)
    y1 = jnp.maximum(y1 + bnum_ref[:], 0.0)
    wout = wout_ref[:]
    acc = jnp.dot(u_ref[:], wout[0:EMB], preferred_element_type=jnp.float32)
    acc += jnp.dot(i_ref[:], wout[EMB:2 * EMB],
                   preferred_element_type=jnp.float32)
    acc += jnp.dot(y1, wout[2 * EMB:3 * EMB],
                   preferred_element_type=jnp.float32)
    o_ref[:] = jnp.maximum(acc + bout_ref[:], 0.0)


def _mlp(u, i, nf_pad, wnum_pad, bnum, wout, bout, block_b=2048):
    grid = (BATCH // block_b,)
    return pl.pallas_call(
        _mlp_body,
        grid=grid,
        in_specs=[
            pl.BlockSpec((block_b, EMB), lambda g: (g, 0)),
            pl.BlockSpec((block_b, EMB), lambda g: (g, 0)),
            pl.BlockSpec((block_b, NF_PAD), lambda g: (g, 0)),
            pl.BlockSpec((NF_PAD, EMB), lambda g: (0, 0)),
            pl.BlockSpec((1, EMB), lambda g: (0, 0)),
            pl.BlockSpec((3 * EMB, EMB), lambda g: (0, 0)),
            pl.BlockSpec((1, EMB), lambda g: (0, 0)),
        ],
        out_specs=pl.BlockSpec((block_b, EMB), lambda g: (g, 0)),
        out_shape=jax.ShapeDtypeStruct((BATCH, EMB), jnp.float32),
    )(u, i, nf_pad, wnum_pad, bnum, wout, bout)


def kernel(inputs, user_table, item_table, W_num, b_num, W_out, b_out):
    user_ids = inputs[:, 0].astype(jnp.int32)
    item_ids = inputs[:, 1].astype(jnp.int32)
    uidx2 = user_ids.reshape(_NW * _ROWS, _L)
    iidx2 = item_ids.reshape(_NW * _ROWS, _L)
    nf_pad = jnp.pad(inputs[:, 2:], ((0, 0), (0, NF_PAD - 5)))
    wnum_pad = jnp.pad(W_num, ((0, NF_PAD - 5), (0, 0)))
    u_rows, i_rows = _sc_gather()(user_table.T, item_table.T, uidx2, iidx2)
    return _mlp(u_rows, i_rows, nf_pad, wnum_pad,
                b_num.reshape(1, EMB), W_out, b_out.reshape(1, EMB))


# speedup vs baseline: 3.1191x; 1.1332x over previous
import functools

import jax
import jax.numpy as jnp
from jax import lax
from jax.experimental import pallas as pl
from jax.experimental.pallas import tpu as pltpu
from jax.experimental.pallas import tpu_sc as plsc

BATCH = 16384
EMB = 64
NF_PAD = 8

_NC, _NS = 2, 16
_NW = _NC * _NS
_BPW = BATCH // _NW
_L = 16
_RING = 4
_ROWS = _BPW // _L


def _gather_blocks(tabT, idx2_v, out_v, ring, sem):
    iota = lax.iota(jnp.int32, _L)
    tabT3 = tabT.reshape(EMB // 8, 8, tabT.shape[1])

    def fire(id_scalar, slot):
        b0 = pl.multiple_of((id_scalar >> 7) * 128, 128)
        pltpu.async_copy(tabT3.at[:, :, pl.ds(b0, 128)], ring.at[slot], sem)

    def wait_slot(slot):
        pltpu.make_async_copy(tabT3.at[:, :, pl.ds(0, 128)], ring.at[slot],
                              sem).wait()

    def extract(id_scalar, pos, slot):
        lane = jnp.full((_L,), id_scalar & 127, jnp.int32)
        buf = ring.at[slot].reshape(EMB, 128)
        for k in range(EMB // _L):
            vals = plsc.load_gather(buf, [iota + _L * k, lane])
            out_v[pos, pl.ds(_L * k, _L)] = vals

    def sid(vec, vecp, lane):
        return vec[lane] if lane >= 0 else vecp[_L + lane]

    def d3_at(vec, vecp, j, q):
        b = {l: sid(vec, vecp, l) >> 7 for l in range(j - 3, j + 1)}
        c1 = (q >= 1) & (b[j] == b[j - 1])
        c1m = (q >= 2) & (b[j - 1] == b[j - 2])
        c1mm = (q >= 3) & (b[j - 2] == b[j - 3])
        ch2 = c1 & c1m
        ch3 = ch2 & c1mm
        return (c1.astype(jnp.int32) + ch2.astype(jnp.int32)
                + ch3.astype(jnp.int32))

    def drain_unit(vec, vecp, j, q):
        slot = j % _RING
        d3p = d3_at(vec, vecp, j - _RING, q - _RING)
        fired_prev = (d3p == 0) | (d3p == 3)

        @pl.when(fired_prev)
        def _():
            wait_slot(slot)

        for dd in range(3):
            d3x = d3_at(vec, vecp, j - _RING + dd, q - _RING + dd)
            take = jnp.where(d3x == 3, 0, d3x) == dd

            @pl.when(take)
            def _():
                extract(sid(vec, vecp, j - _RING + dd), q - _RING + dd, slot)

    def body(w, carry):
        vec = idx2_v[w]
        vecp = idx2_v[jnp.maximum(w - 1, 0)]
        for j in range(_L):
            q = w * _L + j

            @pl.when(q >= _RING)
            def _():
                drain_unit(vec, vecp, j, q)

            d3q = d3_at(vec, vecp, j, q)

            @pl.when((d3q == 0) | (d3q == 3))
            def _():
                fire(sid(vec, vecp, j), j % _RING)
        return carry

    lax.fori_loop(0, _ROWS, body, 0)
    vec_last = idx2_v[_ROWS - 1]
    vec_prev = idx2_v[_ROWS - 2]
    for qq in range(_BPW, _BPW + _RING):
        j = qq - (_ROWS - 1) * _L
        slot = j % _RING
        d3p = d3_at(vec_last, vec_prev, j - _RING, qq - _RING)
        fired_prev = (d3p == 0) | (d3p == 3)

        @pl.when(fired_prev)
        def _():
            wait_slot(slot)

        for dd in range(3):
            p2 = qq - _RING + dd
            if p2 >= _BPW:
                continue
            d3x = d3_at(vec_last, vec_prev, j - _RING + dd, p2)
            take = jnp.where(d3x == 3, 0, d3x) == dd

            @pl.when(take)
            def _():
                extract(sid(vec_last, vec_prev, j - _RING + dd), p2, slot)


def _scatter_rows(out_v, perm2_v, out_hbm, sem):
    def body(w, carry):
        pv = perm2_v[w]
        for j in range(_L):
            dst = pv[j]
            pltpu.async_copy(out_v.at[pl.ds(w * _L + j, 1)],
                             out_hbm.at[pl.ds(dst, 1)], sem)
        return carry

    lax.fori_loop(0, _ROWS, body, 0)
    pltpu.make_async_copy(out_hbm.at[pl.ds(0, _BPW)], out_v, sem).wait()


def _sc_gather_body(utabT, itabT, uidx, iidx, uperm, iperm, uout, iout,
                    uidx2_v, iidx2_v, uperm_v, iperm_v, out_v, ring,
                    sem, semw):
    wid = lax.axis_index("s") * _NC + lax.axis_index("c")
    pltpu.sync_copy(uidx.at[pl.ds(wid * _ROWS, _ROWS)], uidx2_v)
    pltpu.sync_copy(iidx.at[pl.ds(wid * _ROWS, _ROWS)], iidx2_v)
    pltpu.sync_copy(uperm.at[pl.ds(wid * _ROWS, _ROWS)], uperm_v)
    pltpu.sync_copy(iperm.at[pl.ds(wid * _ROWS, _ROWS)], iperm_v)
    _gather_blocks(utabT, uidx2_v, out_v, ring, sem)
    _scatter_rows(out_v, uperm_v, uout, semw)
    _gather_blocks(itabT, iidx2_v, out_v, ring, sem)
    _scatter_rows(out_v, iperm_v, iout, semw)


@functools.cache
def _sc_gather():
    return pl.kernel(
        _sc_gather_body,
        out_type=(
            jax.ShapeDtypeStruct((BATCH, EMB), jnp.float32),
            jax.ShapeDtypeStruct((BATCH, EMB), jnp.float32),
        ),
        mesh=plsc.VectorSubcoreMesh(core_axis_name="c", subcore_axis_name="s",
                                    num_cores=_NC, num_subcores=_NS),
        compiler_params=pltpu.CompilerParams(needs_layout_passes=False),
        scratch_types=[
            pltpu.VMEM((_ROWS, _L), jnp.int32),
            pltpu.VMEM((_ROWS, _L), jnp.int32),
            pltpu.VMEM((_ROWS, _L), jnp.int32),
            pltpu.VMEM((_ROWS, _L), jnp.int32),
            pltpu.VMEM((_BPW, EMB), jnp.float32),
            pltpu.VMEM((_RING, EMB // 8, 8, 128), jnp.float32),
            pltpu.SemaphoreType.DMA,
            pltpu.SemaphoreType.DMA,
        ],
    )


def _mlp_body(u_ref, i_ref, nf_ref, wnum_ref, bnum_ref, wout_ref, bout_ref,
              o_ref):
    y1 = jnp.dot(nf_ref[:], wnum_ref[:], preferred_element_type=jnp.float32)
    y1 = jnp.maximum(y1 + bnum_ref[:], 0.0)
    wout = wout_ref[:]
    acc = jnp.dot(u_ref[:], wout[0:EMB], preferred_element_type=jnp.float32)
    acc += jnp.dot(i_ref[:], wout[EMB:2 * EMB],
                   preferred_element_type=jnp.float32)
    acc += jnp.dot(y1, wout[2 * EMB:3 * EMB],
                   preferred_element_type=jnp.float32)
    o_ref[:] = jnp.maximum(acc + bout_ref[:], 0.0)


def _mlp(u, i, nf_pad, wnum_pad, bnum, wout, bout, block_b=2048):
    grid = (BATCH // block_b,)
    return pl.pallas_call(
        _mlp_body,
        grid=grid,
        in_specs=[
            pl.BlockSpec((block_b, EMB), lambda g: (g, 0)),
            pl.BlockSpec((block_b, EMB), lambda g: (g, 0)),
            pl.BlockSpec((block_b, NF_PAD), lambda g: (g, 0)),
            pl.BlockSpec((NF_PAD, EMB), lambda g: (0, 0)),
            pl.BlockSpec((1, EMB), lambda g: (0, 0)),
            pl.BlockSpec((3 * EMB, EMB), lambda g: (0, 0)),
            pl.BlockSpec((1, EMB), lambda g: (0, 0)),
        ],
        out_specs=pl.BlockSpec((block_b, EMB), lambda g: (g, 0)),
        out_shape=jax.ShapeDtypeStruct((BATCH, EMB), jnp.float32),
    )(u, i, nf_pad, wnum_pad, bnum, wout, bout)


def kernel(inputs, user_table, item_table, W_num, b_num, W_out, b_out):
    user_ids = inputs[:, 0].astype(jnp.int32)
    item_ids = inputs[:, 1].astype(jnp.int32)
    uperm = jnp.argsort(user_ids).astype(jnp.int32)
    iperm = jnp.argsort(item_ids).astype(jnp.int32)
    us = jnp.sort(user_ids).reshape(_NW * _ROWS, _L)
    its = jnp.sort(item_ids).reshape(_NW * _ROWS, _L)
    uperm2 = uperm.reshape(_NW * _ROWS, _L)
    iperm2 = iperm.reshape(_NW * _ROWS, _L)
    nf_pad = jnp.pad(inputs[:, 2:], ((0, 0), (0, NF_PAD - 5)))
    wnum_pad = jnp.pad(W_num, ((0, NF_PAD - 5), (0, 0)))
    u_rows, i_rows = _sc_gather()(user_table.T, item_table.T, us, its,
                                  uperm2, iperm2)
    return _mlp(u_rows, i_rows, nf_pad, wnum_pad,
                b_num.reshape(1, EMB), W_out, b_out.reshape(1, EMB))


# speedup vs baseline: 3.1353x; 1.0052x over previous
import functools

import jax
import jax.numpy as jnp
from jax import lax
from jax.experimental import pallas as pl
from jax.experimental.pallas import tpu as pltpu
from jax.experimental.pallas import tpu_sc as plsc

BATCH = 16384
EMB = 64
NF_PAD = 8

_NC, _NS = 2, 16
_NW = _NC * _NS
_BPW = BATCH // _NW
_L = 16
_RING = 4
_ROWS = _BPW // _L


def _gather_blocks(tabT, idx2_v, out_v, ring, sem):
    iota = lax.iota(jnp.int32, _L)
    tabT3 = tabT.reshape(EMB // 8, 8, tabT.shape[1])

    def fire(id_scalar, slot):
        b0 = pl.multiple_of((id_scalar >> 7) * 128, 128)
        pltpu.async_copy(tabT3.at[:, :, pl.ds(b0, 128)], ring.at[slot], sem)

    def wait_slot(slot):
        pltpu.make_async_copy(tabT3.at[:, :, pl.ds(0, 128)], ring.at[slot],
                              sem).wait()

    def extract(id_scalar, pos, slot):
        lane = jnp.full((_L,), id_scalar & 127, jnp.int32)
        buf = ring.at[slot].reshape(EMB, 128)
        for k in range(EMB // _L):
            vals = plsc.load_gather(buf, [iota + _L * k, lane])
            out_v[pos, pl.ds(_L * k, _L)] = vals

    def sid(vec, vecp, lane):
        return vec[lane] if lane >= 0 else vecp[_L + lane]

    iota16 = lax.iota(jnp.int32, _L)

    def shifted_bid(w, k):
        row = jnp.maximum(w - (iota16 < k).astype(jnp.int32), 0)
        lane = (iota16 - k) & (_L - 1)
        return plsc.load_gather(idx2_v, [row, lane]) >> 7

    def d3_vec(vec, vecp, w):
        bid = vec >> 7
        bm1 = shifted_bid(w, 1)
        bm2 = shifted_bid(w, 2)
        bm3 = shifted_bid(w, 3)
        qv = w * _L + iota16
        c1 = (bid == bm1) & (qv >= 1)
        c1m = (bm1 == bm2) & (qv >= 2)
        c1mm = (bm2 == bm3) & (qv >= 3)
        ch2 = c1 & c1m
        ch3 = ch2 & c1mm
        return (c1.astype(jnp.int32) + ch2.astype(jnp.int32)
                + ch3.astype(jnp.int32))

    def d3_lane(d3_v, d3p_v, lane):
        return d3_v[lane] if lane >= 0 else d3p_v[_L + lane]

    def body(w, carry):
        d3p_v = carry
        vec = idx2_v[w]
        vecp = idx2_v[jnp.maximum(w - 1, 0)]
        d3_v = d3_vec(vec, vecp, w)
        for j in range(_L):
            q = w * _L + j
            slot = j % _RING
            d3p = d3_lane(d3_v, d3p_v, j - _RING)
            fired_prev = (d3p == 0) | (d3p == 3)

            @pl.when((q >= _RING) & fired_prev)
            def _():
                wait_slot(slot)

            for dd in range(3):
                d3x = d3_lane(d3_v, d3p_v, j - _RING + dd)
                take = (jnp.where(d3x == 3, 0, d3x) == dd) & (q >= _RING)

                @pl.when(take)
                def _():
                    extract(sid(vec, vecp, j - _RING + dd),
                            q - _RING + dd, slot)

            d3q = d3_v[j]

            @pl.when((d3q == 0) | (d3q == 3))
            def _():
                fire(vec[j], slot)
        return d3_v

    d3_last = lax.fori_loop(0, _ROWS, body, jnp.zeros((_L,), jnp.int32))
    vec_last = idx2_v[_ROWS - 1]
    vec_prev = idx2_v[_ROWS - 2]
    for qq in range(_BPW, _BPW + _RING):
        j = qq - (_ROWS - 1) * _L
        slot = j % _RING
        d3p = d3_last[j - _RING]
        fired_prev = (d3p == 0) | (d3p == 3)

        @pl.when(fired_prev)
        def _():
            wait_slot(slot)

        for dd in range(3):
            p2 = qq - _RING + dd
            if p2 >= _BPW:
                continue
            d3x = d3_last[j - _RING + dd]
            take = jnp.where(d3x == 3, 0, d3x) == dd

            @pl.when(take)
            def _():
                extract(sid(vec_last, vec_prev, j - _RING + dd), p2, slot)


def _scatter_rows(out_v, perm2_v, out_hbm, sem):
    def body(w, carry):
        pv = perm2_v[w]
        for j in range(_L):
            dst = pv[j]
            pltpu.async_copy(out_v.at[pl.ds(w * _L + j, 1)],
                             out_hbm.at[pl.ds(dst, 1)], sem)
        return carry

    lax.fori_loop(0, _ROWS, body, 0)
    pltpu.make_async_copy(out_hbm.at[pl.ds(0, _BPW)], out_v, sem).wait()


def _sc_gather_body(utabT, itabT, uidx, iidx, uperm, iperm, uout, iout,
                    uidx2_v, iidx2_v, uperm_v, iperm_v, out_v, ring,
                    sem, semw):
    wid = lax.axis_index("s") * _NC + lax.axis_index("c")
    pltpu.sync_copy(uidx.at[pl.ds(wid * _ROWS, _ROWS)], uidx2_v)
    pltpu.sync_copy(iidx.at[pl.ds(wid * _ROWS, _ROWS)], iidx2_v)
    pltpu.sync_copy(uperm.at[pl.ds(wid * _ROWS, _ROWS)], uperm_v)
    pltpu.sync_copy(iperm.at[pl.ds(wid * _ROWS, _ROWS)], iperm_v)
    _gather_blocks(utabT, uidx2_v, out_v, ring, sem)
    _scatter_rows(out_v, uperm_v, uout, semw)
    _gather_blocks(itabT, iidx2_v, out_v, ring, sem)
    _scatter_rows(out_v, iperm_v, iout, semw)


@functools.cache
def _sc_gather():
    return pl.kernel(
        _sc_gather_body,
        out_type=(
            jax.ShapeDtypeStruct((BATCH, EMB), jnp.float32),
            jax.ShapeDtypeStruct((BATCH, EMB), jnp.float32),
        ),
        mesh=plsc.VectorSubcoreMesh(core_axis_name="c", subcore_axis_name="s",
                                    num_cores=_NC, num_subcores=_NS),
        compiler_params=pltpu.CompilerParams(needs_layout_passes=False),
        scratch_types=[
            pltpu.VMEM((_ROWS, _L), jnp.int32),
            pltpu.VMEM((_ROWS, _L), jnp.int32),
            pltpu.VMEM((_ROWS, _L), jnp.int32),
            pltpu.VMEM((_ROWS, _L), jnp.int32),
            pltpu.VMEM((_BPW, EMB), jnp.float32),
            pltpu.VMEM((_RING, EMB // 8, 8, 128), jnp.float32),
            pltpu.SemaphoreType.DMA,
            pltpu.SemaphoreType.DMA,
        ],
    )


def _mlp_body(u_ref, i_ref, nf_ref, wnum_ref, bnum_ref, wout_ref, bout_ref,
              o_ref):
    y1 = jnp.dot(nf_ref[:], wnum_ref[:], preferred_element_type=jnp.float32)
    y1 = jnp.maximum(y1 + bnum_ref[:], 0.0)
    wout = wout_ref[:]
    acc = jnp.dot(u_ref[:], wout[0:EMB], preferred_element_type=jnp.float32)
    acc += jnp.dot(i_ref[:], wout[EMB:2 * EMB],
                   preferred_element_type=jnp.float32)
    acc += jnp.dot(y1, wout[2 * EMB:3 * EMB],
                   preferred_element_type=jnp.float32)
    o_ref[:] = jnp.maximum(acc + bout_ref[:], 0.0)


def _mlp(u, i, nf_pad, wnum_pad, bnum, wout, bout, block_b=2048):
    grid = (BATCH // block_b,)
    return pl.pallas_call(
        _mlp_body,
        grid=grid,
        in_specs=[
            pl.BlockSpec((block_b, EMB), lambda g: (g, 0)),
            pl.BlockSpec((block_b, EMB), lambda g: (g, 0)),
            pl.BlockSpec((block_b, NF_PAD), lambda g: (g, 0)),
            pl.BlockSpec((NF_PAD, EMB), lambda g: (0, 0)),
            pl.BlockSpec((1, EMB), lambda g: (0, 0)),
            pl.BlockSpec((3 * EMB, EMB), lambda g: (0, 0)),
            pl.BlockSpec((1, EMB), lambda g: (0, 0)),
        ],
        out_specs=pl.BlockSpec((block_b, EMB), lambda g: (g, 0)),
        out_shape=jax.ShapeDtypeStruct((BATCH, EMB), jnp.float32),
    )(u, i, nf_pad, wnum_pad, bnum, wout, bout)


def kernel(inputs, user_table, item_table, W_num, b_num, W_out, b_out):
    user_ids = inputs[:, 0].astype(jnp.int32)
    item_ids = inputs[:, 1].astype(jnp.int32)
    uperm = jnp.argsort(user_ids).astype(jnp.int32)
    iperm = jnp.argsort(item_ids).astype(jnp.int32)
    us = jnp.sort(user_ids).reshape(_NW * _ROWS, _L)
    its = jnp.sort(item_ids).reshape(_NW * _ROWS, _L)
    uperm2 = uperm.reshape(_NW * _ROWS, _L)
    iperm2 = iperm.reshape(_NW * _ROWS, _L)
    nf_pad = jnp.pad(inputs[:, 2:], ((0, 0), (0, NF_PAD - 5)))
    wnum_pad = jnp.pad(W_num, ((0, NF_PAD - 5), (0, 0)))
    u_rows, i_rows = _sc_gather()(user_table.T, item_table.T, us, its,
                                  uperm2, iperm2)
    return _mlp(u_rows, i_rows, nf_pad, wnum_pad,
                b_num.reshape(1, EMB), W_out, b_out.reshape(1, EMB))
